# baseline (speedup 1.0000x reference)
import jax
import jax.numpy as jnp
from jax import lax
from jax.experimental import pallas as pl
from jax.experimental.pallas import tpu as pltpu
from jax.experimental.pallas import tpu_sc as plsc

N = 10000
E = 320000
F = 128
M = 128
NW = 32
PER_TILE = E // NW
CHUNK = 80
NCHUNK = PER_TILE // CHUNK
NPAD = 10240
NROW = NPAD // 16
XW = 16


def _silu(v):
    return v / (1.0 + jnp.exp(-v))


def _prep_body(h_ref, we1a_ref, we1b_ref, wn1a_ref, a_ref, b_ref, hn_ref):
    hb = h_ref[...]
    a_ref[...] = jnp.dot(hb, we1a_ref[...], preferred_element_type=jnp.float32)
    b_ref[...] = jnp.dot(hb, we1b_ref[...], preferred_element_type=jnp.float32)
    hn_ref[...] = jnp.dot(hb, wn1a_ref[...], preferred_element_type=jnp.float32)


def _prep(h, we1a, we1b, wn1a):
    bn = 1000
    return pl.pallas_call(
        _prep_body,
        grid=(N // bn,),
        in_specs=[
            pl.BlockSpec((bn, F), lambda i: (i, 0)),
            pl.BlockSpec((F, F), lambda i: (0, 0)),
            pl.BlockSpec((F, F), lambda i: (0, 0)),
            pl.BlockSpec((F, F), lambda i: (0, 0)),
        ],
        out_specs=[
            pl.BlockSpec((bn, F), lambda i: (i, 0)),
            pl.BlockSpec((bn, F), lambda i: (i, 0)),
            pl.BlockSpec((bn, F), lambda i: (i, 0)),
        ],
        out_shape=[
            jax.ShapeDtypeStruct((N, F), jnp.float32),
            jax.ShapeDtypeStruct((N, F), jnp.float32),
            jax.ShapeDtypeStruct((N, F), jnp.float32),
        ],
    )(h, we1a, we1b, wn1a)


DXB = 5


def _gather_body(a_hbm, b_hbm, xpad_hbm, src_hbm, dst_hbm, ef_hbm, dx_hbm,
                 sbuf, dbuf, bufa0, bufa1, bufb0, bufb1, xtab, dxbuf,
                 sema, semb):
    wid = lax.axis_index("s") * 2 + lax.axis_index("c")
    pltpu.sync_copy(src_hbm.at[wid], sbuf)
    pltpu.sync_copy(dst_hbm.at[wid], dbuf)
    pltpu.sync_copy(xpad_hbm, xtab)
    bufas = (bufa0, bufa1)
    bufbs = (bufb0, bufb1)

    def issue(j, k):
        pltpu.async_copy(a_hbm.at[sbuf.at[j]], bufas[k], sema)
        pltpu.async_copy(b_hbm.at[dbuf.at[j]], bufbs[k], semb)

    def wait(k):
        pltpu.make_async_copy(a_hbm.at[sbuf.at[0]], bufas[k], sema).wait()
        pltpu.make_async_copy(b_hbm.at[dbuf.at[0]], bufbs[k], semb).wait()

    issue(0, 0)
    eidx = jnp.arange(16, dtype=jnp.int32)

    def chunk2(j, k, bufa, bufb):
        wait(k)
        issue(jnp.minimum(j + 1, NCHUNK - 1), 1 - k)

        def row(r, c2):
            for c in range(F // 16):
                sl = pl.ds(c * 16, 16)
                bufa[r, sl] = bufa[r, sl] + bufb[r, sl]
            return c2
        lax.fori_loop(0, CHUNK, row, 0)

        def grp(g, c2):
            si = sbuf[j, pl.ds(g * 16, 16)] * 4
            di = dbuf[j, pl.ds(g * 16, 16)] * 4
            ei = ((j % DXB) * CHUNK + g * 16 + eidx) * XW
            for c in range(3):
                xs = plsc.load_gather(xtab, [si + c])
                xd = plsc.load_gather(xtab, [di + c])
                plsc.store_scatter(dxbuf, [ei + c], xd - xs)
            return c2
        lax.fori_loop(0, CHUNK // 16, grp, 0)
        pltpu.sync_copy(bufa, ef_hbm.at[pl.ds(wid * PER_TILE + j * CHUNK, CHUNK)])

        @pl.when(j % DXB == DXB - 1)
        def _():
            jb = j - (DXB - 1)
            pltpu.sync_copy(
                dxbuf,
                dx_hbm.at[pl.ds((wid * PER_TILE + jb * CHUNK) * XW,
                                DXB * CHUNK * XW)])

    def chunk(j, carry):
        @pl.when(j % 2 == 0)
        def _():
            chunk2(j, 0, bufa0, bufb0)

        @pl.when(j % 2 == 1)
        def _():
            chunk2(j, 1, bufa1, bufb1)
        return carry
    lax.fori_loop(0, NCHUNK, chunk, 0)
    wait(NCHUNK % 2)


def _gather(a_tab, b_tab, xpad, src3d, dst3d):
    mesh = plsc.VectorSubcoreMesh(core_axis_name="c", subcore_axis_name="s")
    fn = pl.kernel(
        _gather_body,
        out_type=[
            jax.ShapeDtypeStruct((E, F), jnp.float32),
            jax.ShapeDtypeStruct((E * XW,), jnp.float32),
        ],
        mesh=mesh,
        scratch_types=[
            pltpu.VMEM((NCHUNK, CHUNK), jnp.int32),
            pltpu.VMEM((NCHUNK, CHUNK), jnp.int32),
            pltpu.VMEM((CHUNK, F), jnp.float32),
            pltpu.VMEM((CHUNK, F), jnp.float32),
            pltpu.VMEM((CHUNK, F), jnp.float32),
            pltpu.VMEM((CHUNK, F), jnp.float32),
            pltpu.VMEM((N * 4,), jnp.float32),
            pltpu.VMEM((DXB * CHUNK * XW,), jnp.float32),
            pltpu.SemaphoreType.DMA,
            pltpu.SemaphoreType.DMA,
        ],
        compiler_params=pltpu.CompilerParams(needs_layout_passes=False),
    )
    return fn(a_tab, b_tab, xpad, src3d, dst3d)


def _edge_body(ef_ref, dx_ref, we1d_ref, be1_ref, we2_ref, be2_ref,
               ww1_ref, bw1_ref, ww2_ref, m_ref, xo_ref):
    ef = ef_ref[...]
    dxb = dx_ref[...]
    be = ef.shape[0]
    dxyz = dxb[:, :3]
    d = jnp.sqrt(jnp.sum(dxyz * dxyz, axis=1, keepdims=True))
    pre1 = ef + d * we1d_ref[...] + be1_ref[...]
    h1 = _silu(pre1)
    m = _silu(jnp.dot(h1, we2_ref[...], preferred_element_type=jnp.float32)
              + be2_ref[...])
    t = _silu(jnp.dot(m, ww1_ref[...], preferred_element_type=jnp.float32)
              + bw1_ref[...])
    wgt = jnp.sum(t * ww2_ref[:, :F], axis=1, keepdims=True) + ww2_ref[:, F:F + 1]
    x_ij = dxyz * wgt
    ones = jnp.ones((be, 1), dtype=jnp.float32)
    zpad = jnp.zeros((be, XW - 4), dtype=jnp.float32)
    m_ref[...] = m
    xo_ref[...] = jnp.concatenate([x_ij, ones, zpad], axis=1)


def _edge_mlp(ef, dx, we1d, be1, we2, be2, ww1, bw1, ww2_plus):
    be = 512
    return pl.pallas_call(
        _edge_body,
        grid=(E // be,),
        in_specs=[
            pl.BlockSpec((be, F), lambda i: (i, 0)),
            pl.BlockSpec((be, XW), lambda i: (i, 0)),
            pl.BlockSpec((1, F), lambda i: (0, 0)),
            pl.BlockSpec((1, F), lambda i: (0, 0)),
            pl.BlockSpec((F, F), lambda i: (0, 0)),
            pl.BlockSpec((1, F), lambda i: (0, 0)),
            pl.BlockSpec((F, F), lambda i: (0, 0)),
            pl.BlockSpec((1, F), lambda i: (0, 0)),
            pl.BlockSpec((1, F + 1), lambda i: (0, 0)),
        ],
        out_specs=[
            pl.BlockSpec((be, F), lambda i: (i, 0)),
            pl.BlockSpec((be, XW), lambda i: (i, 0)),
        ],
        out_shape=[
            jax.ShapeDtypeStruct((E, F), jnp.float32),
            jax.ShapeDtypeStruct((E, XW), jnp.float32),
        ],
    )(ef, dx, we1d, be1, we2, be2, ww1, bw1, ww2_plus)


def _scatter_m_body(m_hbm, src_hbm, accm_hbm, sbuf, bufm0, bufm1, accm,
                    seml, sems):
    cid = lax.axis_index("c")
    sid = lax.axis_index("s")
    wid = sid * 2 + cid
    zero16 = jnp.zeros((16,), jnp.float32)
    bufs = (bufm0, bufm1)

    def zrow(r, c2):
        for c in range(F // 16):
            bufm0[r, pl.ds(c * 16, 16)] = zero16
        return c2
    lax.fori_loop(0, CHUNK, zrow, 0)
    for k in range(NROW // CHUNK):
        pltpu.sync_copy(bufm0, accm.at[pl.ds(sid * NROW + k * CHUNK, CHUNK)])
    plsc.subcore_barrier()

    pltpu.sync_copy(src_hbm.at[wid], sbuf)

    def issue(j, k):
        pltpu.async_copy(m_hbm.at[pl.ds(wid * PER_TILE + j * CHUNK, CHUNK)],
                         bufs[k], seml)

    def wait(k):
        pltpu.make_async_copy(m_hbm.at[pl.ds(0, CHUNK)], bufs[k], seml).wait()

    def wait_scat(j, k):
        pltpu.make_async_copy(bufs[k], accm.at[sbuf.at[j]], sems).wait()

    issue(0, 0)

    def chunk2(j, k, bufm):
        wait(k)

        @pl.when(j >= 1)
        def _():
            wait_scat(j - 1, 1 - k)
        issue(jnp.minimum(j + 1, NCHUNK - 1), 1 - k)
        pltpu.async_copy(bufm, accm.at[sbuf.at[j]], sems, add=True)

    def chunk(j, carry):
        @pl.when(j % 2 == 0)
        def _():
            chunk2(j, 0, bufm0)

        @pl.when(j % 2 == 1)
        def _():
            chunk2(j, 1, bufm1)
        return carry
    lax.fori_loop(0, NCHUNK, chunk, 0)
    wait(NCHUNK % 2)
    wait_scat(NCHUNK - 1, (NCHUNK - 1) % 2)
    plsc.subcore_barrier()

    r0 = sid * NROW
    pltpu.sync_copy(accm.at[pl.ds(r0, NROW)], accm_hbm.at[cid].at[pl.ds(r0, NROW)])


def _scatter_m(m_rows, src3d):
    mesh = plsc.VectorSubcoreMesh(core_axis_name="c", subcore_axis_name="s")
    fn = pl.kernel(
        _scatter_m_body,
        out_type=jax.ShapeDtypeStruct((2, NPAD, F), jnp.float32),
        mesh=mesh,
        scratch_types=[
            pltpu.VMEM((NCHUNK, CHUNK), jnp.int32),
            pltpu.VMEM((CHUNK, F), jnp.float32),
            pltpu.VMEM((CHUNK, F), jnp.float32),
            pltpu.VMEM_SHARED((NPAD, F), jnp.float32),
            pltpu.SemaphoreType.DMA,
            pltpu.SemaphoreType.DMA,
        ],
        compiler_params=pltpu.CompilerParams(needs_layout_passes=False),
    )
    return fn(m_rows, src3d)


NXB = 5
NBATCH = NCHUNK // NXB


def _scatter_x_body(xo_hbm, src_hbm, accx_hbm, sbuf, bufx0, bufx1, accx, seml):
    wid = lax.axis_index("s") * 2 + lax.axis_index("c")
    zero16 = jnp.zeros((16,), jnp.float32)
    bufs = (bufx0, bufx1)

    def zx(i, c2):
        accx[pl.ds(i * 16, 16)] = zero16
        return c2
    lax.fori_loop(0, NPAD * 4 // 16, zx, 0)

    pltpu.sync_copy(src_hbm.at[wid], sbuf)
    eidx = jnp.arange(16, dtype=jnp.int32)

    def issue(b, k):
        base = wid * PER_TILE + b * NXB * CHUNK
        pltpu.async_copy(xo_hbm.at[pl.ds(base * XW, NXB * CHUNK * XW)],
                         bufs[k], seml)

    def wait(k):
        pltpu.make_async_copy(xo_hbm.at[pl.ds(0, NXB * CHUNK * XW)],
                              bufs[k], seml).wait()

    issue(0, 0)

    def batch2(b, k, bufx):
        wait(k)
        issue(jnp.minimum(b + 1, NBATCH - 1), 1 - k)

        def grp(g, c2):
            j = b * NXB + g // (CHUNK // 16)
            gg = g % (CHUNK // 16)
            ni = sbuf[j, pl.ds(gg * 16, 16)] * 4
            ei = (g * 16 + eidx) * XW
            for c in range(4):
                v = plsc.load_gather(bufx, [ei + c])
                plsc.addupdate_scatter(accx, [ni + c], v)
            return c2
        lax.fori_loop(0, NXB * CHUNK // 16, grp, 0)

    def batch(b, carry):
        @pl.when(b % 2 == 0)
        def _():
            batch2(b, 0, bufx0)

        @pl.when(b % 2 == 1)
        def _():
            batch2(b, 1, bufx1)
        return carry
    lax.fori_loop(0, NBATCH, batch, 0)
    wait(NBATCH % 2)
    pltpu.sync_copy(accx, accx_hbm.at[wid])


def _scatter_x(xo_flat, src3d):
    mesh = plsc.VectorSubcoreMesh(core_axis_name="c", subcore_axis_name="s")
    fn = pl.kernel(
        _scatter_x_body,
        out_type=jax.ShapeDtypeStruct((NW, NPAD * 4), jnp.float32),
        mesh=mesh,
        scratch_types=[
            pltpu.VMEM((NCHUNK, CHUNK), jnp.int32),
            pltpu.VMEM((NXB * CHUNK * XW,), jnp.float32),
            pltpu.VMEM((NXB * CHUNK * XW,), jnp.float32),
            pltpu.VMEM((NPAD * 4,), jnp.float32),
            pltpu.SemaphoreType.DMA,
        ],
        compiler_params=pltpu.CompilerParams(needs_layout_passes=False),
    )
    return fn(xo_flat, src3d)


def _node_body(accm_ref, accx_ref, x_ref, h_ref, hn_ref, wn1b_ref, bn1_ref,
               wn2_ref, bn2_ref, xp_ref, hp_ref):
    am = accm_ref[...]
    m_sum = am[0] + am[1]
    ax = jnp.sum(accx_ref[...], axis=0)
    x_sum = ax[:, :3]
    cnt = ax[:, 3:4]
    m_i = m_sum / jnp.maximum(cnt, 1.0)
    pre = (hn_ref[...]
           + jnp.dot(m_i, wn1b_ref[...], preferred_element_type=jnp.float32)
           + bn1_ref[...])
    upd = jnp.dot(_silu(pre), wn2_ref[...], preferred_element_type=jnp.float32)
    hp_ref[...] = h_ref[...] + upd + bn2_ref[...]
    xp_ref[...] = x_ref[...] + x_sum


def _node(accm, accx4, x, h, hn, wn1b, bn1, wn2, bn2):
    bn = 1000
    return pl.pallas_call(
        _node_body,
        grid=(N // bn,),
        in_specs=[
            pl.BlockSpec((2, bn, F), lambda i: (0, i, 0)),
            pl.BlockSpec((NW, bn, 4), lambda i: (0, i, 0)),
            pl.BlockSpec((bn, 3), lambda i: (i, 0)),
            pl.BlockSpec((bn, F), lambda i: (i, 0)),
            pl.BlockSpec((bn, F), lambda i: (i, 0)),
            pl.BlockSpec((F, F), lambda i: (0, 0)),
            pl.BlockSpec((1, F), lambda i: (0, 0)),
            pl.BlockSpec((F, F), lambda i: (0, 0)),
            pl.BlockSpec((1, F), lambda i: (0, 0)),
        ],
        out_specs=[
            pl.BlockSpec((bn, 3), lambda i: (i, 0)),
            pl.BlockSpec((bn, F), lambda i: (i, 0)),
        ],
        out_shape=[
            jax.ShapeDtypeStruct((N, 3), jnp.float32),
            jax.ShapeDtypeStruct((N, F), jnp.float32),
        ],
    )(accm, accx4, x, h, hn, wn1b, bn1, wn2, bn2)


def kernel(x, h, edges, We1, be1, We2, be2, Ww1, bw1, Ww2, bw2, Wn1, bn1, Wn2, bn2):
    src3d = edges[0].astype(jnp.int32).reshape(NW, NCHUNK, CHUNK)
    dst3d = edges[1].astype(jnp.int32).reshape(NW, NCHUNK, CHUNK)
    xpad = jnp.pad(x, ((0, 0), (0, 1))).reshape(N * 4)
    we1a = We1[:F]
    we1b = We1[F:2 * F]
    we1d = We1[2 * F:2 * F + 1]
    wn1a = Wn1[:F]
    wn1b = Wn1[F:]
    ww2_plus = jnp.concatenate([Ww2.reshape(1, F), bw2.reshape(1, 1)], axis=1)

    a_tab, b_tab, hn = _prep(h, we1a, we1b, wn1a)
    ef, dx = _gather(a_tab, b_tab, xpad, src3d, dst3d)
    m_rows, xo_rows = _edge_mlp(ef, dx.reshape(E, XW), we1d,
                                be1.reshape(1, F), We2,
                                be2.reshape(1, F), Ww1, bw1.reshape(1, F),
                                ww2_plus)
    accm = _scatter_m(m_rows, src3d)
    accx = _scatter_x(xo_rows.reshape(E * XW), src3d)
    accx4 = accx.reshape(NW, NPAD, 4)
    x_prime, h_prime = _node(accm, accx4, x, h, hn, wn1b, bn1.reshape(1, F),
                             Wn2, bn2.reshape(1, F))
    return (x_prime, h_prime)


# speedup vs baseline: 1.0673x; 1.0673x over previous
import jax
import jax.numpy as jnp
from jax import lax
from jax.experimental import pallas as pl
from jax.experimental.pallas import tpu as pltpu
from jax.experimental.pallas import tpu_sc as plsc

N = 10000
E = 320000
F = 128
M = 128
NW = 32
PER_TILE = E // NW
CHUNK = 80
NCHUNK = PER_TILE // CHUNK
NPAD = 10240
NROW = NPAD // 16
XW = 16


def _silu(v):
    return v / (1.0 + jnp.exp(-v))


def _prep_body(h_ref, we1a_ref, we1b_ref, wn1a_ref, a_ref, b_ref, hn_ref):
    hb = h_ref[...]
    a_ref[...] = jnp.dot(hb, we1a_ref[...], preferred_element_type=jnp.float32)
    b_ref[...] = jnp.dot(hb, we1b_ref[...], preferred_element_type=jnp.float32)
    hn_ref[...] = jnp.dot(hb, wn1a_ref[...], preferred_element_type=jnp.float32)


def _prep(h, we1a, we1b, wn1a):
    bn = 1000
    return pl.pallas_call(
        _prep_body,
        grid=(N // bn,),
        in_specs=[
            pl.BlockSpec((bn, F), lambda i: (i, 0)),
            pl.BlockSpec((F, F), lambda i: (0, 0)),
            pl.BlockSpec((F, F), lambda i: (0, 0)),
            pl.BlockSpec((F, F), lambda i: (0, 0)),
        ],
        out_specs=[
            pl.BlockSpec((bn, F), lambda i: (i, 0)),
            pl.BlockSpec((bn, F), lambda i: (i, 0)),
            pl.BlockSpec((bn, F), lambda i: (i, 0)),
        ],
        out_shape=[
            jax.ShapeDtypeStruct((N, F), jnp.float32),
            jax.ShapeDtypeStruct((N, F), jnp.float32),
            jax.ShapeDtypeStruct((N, F), jnp.float32),
        ],
    )(h, we1a, we1b, wn1a)


DXB = 5


def _gather_body(a_hbm, b_hbm, xpad_hbm, src_hbm, dst_hbm, ef_hbm, dx_hbm,
                 sbuf, dbuf, bufa0, bufa1, bufb0, bufb1, xtab, dxbuf,
                 sema, semb):
    wid = lax.axis_index("s") * 2 + lax.axis_index("c")
    pltpu.sync_copy(src_hbm.at[wid], sbuf)
    pltpu.sync_copy(dst_hbm.at[wid], dbuf)
    pltpu.sync_copy(xpad_hbm, xtab)
    bufas = (bufa0, bufa1)
    bufbs = (bufb0, bufb1)

    def issue(j, k):
        pltpu.async_copy(a_hbm.at[sbuf.at[j]], bufas[k], sema)
        pltpu.async_copy(b_hbm.at[dbuf.at[j]], bufbs[k], semb)

    def wait(k):
        pltpu.make_async_copy(a_hbm.at[sbuf.at[0]], bufas[k], sema).wait()
        pltpu.make_async_copy(b_hbm.at[dbuf.at[0]], bufbs[k], semb).wait()

    issue(0, 0)
    eidx = jnp.arange(16, dtype=jnp.int32)

    def chunk2(j, k, bufa, bufb):
        wait(k)
        issue(jnp.minimum(j + 1, NCHUNK - 1), 1 - k)

        def row(r, c2):
            for c in range(F // 16):
                sl = pl.ds(c * 16, 16)
                bufa[r, sl] = bufa[r, sl] + bufb[r, sl]
            return c2
        lax.fori_loop(0, CHUNK, row, 0)

        def grp(g, c2):
            si = sbuf[j, pl.ds(g * 16, 16)] * 4
            di = dbuf[j, pl.ds(g * 16, 16)] * 4
            ei = ((j % DXB) * CHUNK + g * 16 + eidx) * XW
            for c in range(3):
                xs = plsc.load_gather(xtab, [si + c])
                xd = plsc.load_gather(xtab, [di + c])
                plsc.store_scatter(dxbuf, [ei + c], xd - xs)
            return c2
        lax.fori_loop(0, CHUNK // 16, grp, 0)
        pltpu.sync_copy(bufa, ef_hbm.at[pl.ds(wid * PER_TILE + j * CHUNK, CHUNK)])

        @pl.when(j % DXB == DXB - 1)
        def _():
            jb = j - (DXB - 1)
            pltpu.sync_copy(
                dxbuf,
                dx_hbm.at[pl.ds((wid * PER_TILE + jb * CHUNK) * XW,
                                DXB * CHUNK * XW)])

    def chunk(j, carry):
        @pl.when(j % 2 == 0)
        def _():
            chunk2(j, 0, bufa0, bufb0)

        @pl.when(j % 2 == 1)
        def _():
            chunk2(j, 1, bufa1, bufb1)
        return carry
    lax.fori_loop(0, NCHUNK, chunk, 0)
    wait(NCHUNK % 2)


def _gather(a_tab, b_tab, xpad, src3d, dst3d):
    mesh = plsc.VectorSubcoreMesh(core_axis_name="c", subcore_axis_name="s")
    fn = pl.kernel(
        _gather_body,
        out_type=[
            jax.ShapeDtypeStruct((E, F), jnp.float32),
            jax.ShapeDtypeStruct((E * XW,), jnp.float32),
        ],
        mesh=mesh,
        scratch_types=[
            pltpu.VMEM((NCHUNK, CHUNK), jnp.int32),
            pltpu.VMEM((NCHUNK, CHUNK), jnp.int32),
            pltpu.VMEM((CHUNK, F), jnp.float32),
            pltpu.VMEM((CHUNK, F), jnp.float32),
            pltpu.VMEM((CHUNK, F), jnp.float32),
            pltpu.VMEM((CHUNK, F), jnp.float32),
            pltpu.VMEM((N * 4,), jnp.float32),
            pltpu.VMEM((DXB * CHUNK * XW,), jnp.float32),
            pltpu.SemaphoreType.DMA,
            pltpu.SemaphoreType.DMA,
        ],
        compiler_params=pltpu.CompilerParams(needs_layout_passes=False),
    )
    return fn(a_tab, b_tab, xpad, src3d, dst3d)


def _edge_body(ef_ref, dx_ref, we1d_ref, be1_ref, we2_ref, be2_ref,
               ww1_ref, bw1_ref, ww2_ref, m_ref, xo_ref):
    ef = ef_ref[...]
    dxb = dx_ref[...]
    be = ef.shape[0]
    dxyz = dxb[:, :3]
    d = jnp.sqrt(jnp.sum(dxyz * dxyz, axis=1, keepdims=True))
    pre1 = ef + d * we1d_ref[...] + be1_ref[...]
    h1 = _silu(pre1)
    m = _silu(jnp.dot(h1, we2_ref[...], preferred_element_type=jnp.float32)
              + be2_ref[...])
    t = _silu(jnp.dot(m, ww1_ref[...], preferred_element_type=jnp.float32)
              + bw1_ref[...])
    wgt = jnp.sum(t * ww2_ref[:, :F], axis=1, keepdims=True) + ww2_ref[:, F:F + 1]
    x_ij = dxyz * wgt
    ones = jnp.ones((be, 1), dtype=jnp.float32)
    zpad = jnp.zeros((be, XW - 4), dtype=jnp.float32)
    m_ref[...] = m
    xo_ref[...] = jnp.concatenate([x_ij, ones, zpad], axis=1)


def _edge_mlp(ef, dx, we1d, be1, we2, be2, ww1, bw1, ww2_plus):
    be = 1280
    return pl.pallas_call(
        _edge_body,
        grid=(E // be,),
        in_specs=[
            pl.BlockSpec((be, F), lambda i: (i, 0)),
            pl.BlockSpec((be, XW), lambda i: (i, 0)),
            pl.BlockSpec((1, F), lambda i: (0, 0)),
            pl.BlockSpec((1, F), lambda i: (0, 0)),
            pl.BlockSpec((F, F), lambda i: (0, 0)),
            pl.BlockSpec((1, F), lambda i: (0, 0)),
            pl.BlockSpec((F, F), lambda i: (0, 0)),
            pl.BlockSpec((1, F), lambda i: (0, 0)),
            pl.BlockSpec((1, F + 1), lambda i: (0, 0)),
        ],
        out_specs=[
            pl.BlockSpec((be, F), lambda i: (i, 0)),
            pl.BlockSpec((be, XW), lambda i: (i, 0)),
        ],
        out_shape=[
            jax.ShapeDtypeStruct((E, F), jnp.float32),
            jax.ShapeDtypeStruct((E, XW), jnp.float32),
        ],
    )(ef, dx, we1d, be1, we2, be2, ww1, bw1, ww2_plus)


def _scatter_m_body(m_hbm, src_hbm, accm_hbm, sbuf, bufm0, bufm1, accm,
                    seml, sems):
    cid = lax.axis_index("c")
    sid = lax.axis_index("s")
    wid = sid * 2 + cid
    zero16 = jnp.zeros((16,), jnp.float32)
    bufs = (bufm0, bufm1)

    def zrow(r, c2):
        for c in range(F // 16):
            bufm0[r, pl.ds(c * 16, 16)] = zero16
        return c2
    lax.fori_loop(0, CHUNK, zrow, 0)
    for k in range(NROW // CHUNK):
        pltpu.sync_copy(bufm0, accm.at[pl.ds(sid * NROW + k * CHUNK, CHUNK)])
    plsc.subcore_barrier()

    pltpu.sync_copy(src_hbm.at[wid], sbuf)

    def issue(j, k):
        pltpu.async_copy(m_hbm.at[pl.ds(wid * PER_TILE + j * CHUNK, CHUNK)],
                         bufs[k], seml)

    def wait(k):
        pltpu.make_async_copy(m_hbm.at[pl.ds(0, CHUNK)], bufs[k], seml).wait()

    def wait_scat(j, k):
        pltpu.make_async_copy(bufs[k], accm.at[sbuf.at[j]], sems).wait()

    issue(0, 0)

    def chunk2(j, k, bufm):
        wait(k)

        @pl.when(j >= 1)
        def _():
            wait_scat(j - 1, 1 - k)
        issue(jnp.minimum(j + 1, NCHUNK - 1), 1 - k)
        pltpu.async_copy(bufm, accm.at[sbuf.at[j]], sems, add=True)

    def chunk(j, carry):
        @pl.when(j % 2 == 0)
        def _():
            chunk2(j, 0, bufm0)

        @pl.when(j % 2 == 1)
        def _():
            chunk2(j, 1, bufm1)
        return carry
    lax.fori_loop(0, NCHUNK, chunk, 0)
    wait(NCHUNK % 2)
    wait_scat(NCHUNK - 1, (NCHUNK - 1) % 2)
    plsc.subcore_barrier()

    r0 = sid * NROW
    pltpu.sync_copy(accm.at[pl.ds(r0, NROW)], accm_hbm.at[cid].at[pl.ds(r0, NROW)])


def _scatter_m(m_rows, src3d):
    mesh = plsc.VectorSubcoreMesh(core_axis_name="c", subcore_axis_name="s")
    fn = pl.kernel(
        _scatter_m_body,
        out_type=jax.ShapeDtypeStruct((2, NPAD, F), jnp.float32),
        mesh=mesh,
        scratch_types=[
            pltpu.VMEM((NCHUNK, CHUNK), jnp.int32),
            pltpu.VMEM((CHUNK, F), jnp.float32),
            pltpu.VMEM((CHUNK, F), jnp.float32),
            pltpu.VMEM_SHARED((NPAD, F), jnp.float32),
            pltpu.SemaphoreType.DMA,
            pltpu.SemaphoreType.DMA,
        ],
        compiler_params=pltpu.CompilerParams(needs_layout_passes=False),
    )
    return fn(m_rows, src3d)


NXB = 5
NBATCH = NCHUNK // NXB


def _scatter_x_body(xo_hbm, src_hbm, accx_hbm, sbuf, bufx0, bufx1, accx, seml):
    wid = lax.axis_index("s") * 2 + lax.axis_index("c")
    zero16 = jnp.zeros((16,), jnp.float32)
    bufs = (bufx0, bufx1)

    def zx(i, c2):
        accx[pl.ds(i * 16, 16)] = zero16
        return c2
    lax.fori_loop(0, NPAD * 4 // 16, zx, 0)

    pltpu.sync_copy(src_hbm.at[wid], sbuf)
    eidx = jnp.arange(16, dtype=jnp.int32)

    def issue(b, k):
        base = wid * PER_TILE + b * NXB * CHUNK
        pltpu.async_copy(xo_hbm.at[pl.ds(base * XW, NXB * CHUNK * XW)],
                         bufs[k], seml)

    def wait(k):
        pltpu.make_async_copy(xo_hbm.at[pl.ds(0, NXB * CHUNK * XW)],
                              bufs[k], seml).wait()

    issue(0, 0)

    def batch2(b, k, bufx):
        wait(k)
        issue(jnp.minimum(b + 1, NBATCH - 1), 1 - k)

        def grp(g, c2):
            j = b * NXB + g // (CHUNK // 16)
            gg = g % (CHUNK // 16)
            ni = sbuf[j, pl.ds(gg * 16, 16)] * 4
            ei = (g * 16 + eidx) * XW
            for c in range(4):
                v = plsc.load_gather(bufx, [ei + c])
                plsc.addupdate_scatter(accx, [ni + c], v)
            return c2
        lax.fori_loop(0, NXB * CHUNK // 16, grp, 0)

    def batch(b, carry):
        @pl.when(b % 2 == 0)
        def _():
            batch2(b, 0, bufx0)

        @pl.when(b % 2 == 1)
        def _():
            batch2(b, 1, bufx1)
        return carry
    lax.fori_loop(0, NBATCH, batch, 0)
    wait(NBATCH % 2)
    pltpu.sync_copy(accx, accx_hbm.at[wid])


def _scatter_x(xo_flat, src3d):
    mesh = plsc.VectorSubcoreMesh(core_axis_name="c", subcore_axis_name="s")
    fn = pl.kernel(
        _scatter_x_body,
        out_type=jax.ShapeDtypeStruct((NW, NPAD * 4), jnp.float32),
        mesh=mesh,
        scratch_types=[
            pltpu.VMEM((NCHUNK, CHUNK), jnp.int32),
            pltpu.VMEM((NXB * CHUNK * XW,), jnp.float32),
            pltpu.VMEM((NXB * CHUNK * XW,), jnp.float32),
            pltpu.VMEM((NPAD * 4,), jnp.float32),
            pltpu.SemaphoreType.DMA,
        ],
        compiler_params=pltpu.CompilerParams(needs_layout_passes=False),
    )
    return fn(xo_flat, src3d)


def _node_body(accm_ref, accx_ref, x_ref, h_ref, hn_ref, wn1b_ref, bn1_ref,
               wn2_ref, bn2_ref, xp_ref, hp_ref):
    am = accm_ref[...]
    m_sum = am[0] + am[1]
    ax = jnp.sum(accx_ref[...], axis=0)
    x_sum = ax[:, :3]
    cnt = ax[:, 3:4]
    m_i = m_sum / jnp.maximum(cnt, 1.0)
    pre = (hn_ref[...]
           + jnp.dot(m_i, wn1b_ref[...], preferred_element_type=jnp.float32)
           + bn1_ref[...])
    upd = jnp.dot(_silu(pre), wn2_ref[...], preferred_element_type=jnp.float32)
    hp_ref[...] = h_ref[...] + upd + bn2_ref[...]
    xp_ref[...] = x_ref[...] + x_sum


def _node(accm, accx4, x, h, hn, wn1b, bn1, wn2, bn2):
    bn = 1000
    return pl.pallas_call(
        _node_body,
        grid=(N // bn,),
        in_specs=[
            pl.BlockSpec((2, bn, F), lambda i: (0, i, 0)),
            pl.BlockSpec((NW, bn, 4), lambda i: (0, i, 0)),
            pl.BlockSpec((bn, 3), lambda i: (i, 0)),
            pl.BlockSpec((bn, F), lambda i: (i, 0)),
            pl.BlockSpec((bn, F), lambda i: (i, 0)),
            pl.BlockSpec((F, F), lambda i: (0, 0)),
            pl.BlockSpec((1, F), lambda i: (0, 0)),
            pl.BlockSpec((F, F), lambda i: (0, 0)),
            pl.BlockSpec((1, F), lambda i: (0, 0)),
        ],
        out_specs=[
            pl.BlockSpec((bn, 3), lambda i: (i, 0)),
            pl.BlockSpec((bn, F), lambda i: (i, 0)),
        ],
        out_shape=[
            jax.ShapeDtypeStruct((N, 3), jnp.float32),
            jax.ShapeDtypeStruct((N, F), jnp.float32),
        ],
    )(accm, accx4, x, h, hn, wn1b, bn1, wn2, bn2)


def kernel(x, h, edges, We1, be1, We2, be2, Ww1, bw1, Ww2, bw2, Wn1, bn1, Wn2, bn2):
    src3d = edges[0].astype(jnp.int32).reshape(NW, NCHUNK, CHUNK)
    dst3d = edges[1].astype(jnp.int32).reshape(NW, NCHUNK, CHUNK)
    xpad = jnp.pad(x, ((0, 0), (0, 1))).reshape(N * 4)
    we1a = We1[:F]
    we1b = We1[F:2 * F]
    we1d = We1[2 * F:2 * F + 1]
    wn1a = Wn1[:F]
    wn1b = Wn1[F:]
    ww2_plus = jnp.concatenate([Ww2.reshape(1, F), bw2.reshape(1, 1)], axis=1)

    a_tab, b_tab, hn = _prep(h, we1a, we1b, wn1a)
    ef, dx = _gather(a_tab, b_tab, xpad, src3d, dst3d)
    m_rows, xo_rows = _edge_mlp(ef, dx.reshape(E, XW), we1d,
                                be1.reshape(1, F), We2,
                                be2.reshape(1, F), Ww1, bw1.reshape(1, F),
                                ww2_plus)
    accm = _scatter_m(m_rows, src3d)
    accx = _scatter_x(xo_rows.reshape(E * XW), src3d)
    accx4 = accx.reshape(NW, NPAD, 4)
    x_prime, h_prime = _node(accm, accx4, x, h, hn, wn1b, bn1.reshape(1, F),
                             Wn2, bn2.reshape(1, F))
    return (x_prime, h_prime)


# speedup vs baseline: 1.7124x; 1.6045x over previous
import jax
import jax.numpy as jnp
from jax import lax
from jax.experimental import pallas as pl
from jax.experimental.pallas import tpu as pltpu
from jax.experimental.pallas import tpu_sc as plsc

N = 10000
E = 320000
F = 128
M = 128
NW = 32
PER_TILE = E // NW
CHUNK = 80
NCHUNK = PER_TILE // CHUNK
NPAD = 10240
NROW = NPAD // 16
XW = 16


def _silu(v):
    return v / (1.0 + jnp.exp(-v))


def _prep_body(h_ref, we1a_ref, we1b_ref, wn1a_ref, a_ref, b_ref, hn_ref):
    hb = h_ref[...]
    a_ref[...] = jnp.dot(hb, we1a_ref[...], preferred_element_type=jnp.float32)
    b_ref[...] = jnp.dot(hb, we1b_ref[...], preferred_element_type=jnp.float32)
    hn_ref[...] = jnp.dot(hb, wn1a_ref[...], preferred_element_type=jnp.float32)


def _prep(h, we1a, we1b, wn1a):
    bn = 1000
    return pl.pallas_call(
        _prep_body,
        grid=(N // bn,),
        in_specs=[
            pl.BlockSpec((bn, F), lambda i: (i, 0)),
            pl.BlockSpec((F, F), lambda i: (0, 0)),
            pl.BlockSpec((F, F), lambda i: (0, 0)),
            pl.BlockSpec((F, F), lambda i: (0, 0)),
        ],
        out_specs=[
            pl.BlockSpec((bn, F), lambda i: (i, 0)),
            pl.BlockSpec((bn, F), lambda i: (i, 0)),
            pl.BlockSpec((bn, F), lambda i: (i, 0)),
        ],
        out_shape=[
            jax.ShapeDtypeStruct((N, F), jnp.float32),
            jax.ShapeDtypeStruct((N, F), jnp.float32),
            jax.ShapeDtypeStruct((N, F), jnp.float32),
        ],
    )(h, we1a, we1b, wn1a)


DXB = 5


def _gather_body(a_hbm, b_hbm, xpad_hbm, src_hbm, dst_hbm, ef_hbm, dx_hbm,
                 sbuf, dbuf, bufa0, bufa1, bufb0, bufb1, xtab, dxbuf,
                 sema, semb):
    wid = lax.axis_index("s") * 2 + lax.axis_index("c")
    pltpu.sync_copy(src_hbm.at[wid], sbuf)
    pltpu.sync_copy(dst_hbm.at[wid], dbuf)
    pltpu.sync_copy(xpad_hbm, xtab)
    bufas = (bufa0, bufa1)
    bufbs = (bufb0, bufb1)

    def issue(j, k):
        pltpu.async_copy(a_hbm.at[sbuf.at[j]], bufas[k], sema)
        pltpu.async_copy(b_hbm.at[dbuf.at[j]], bufbs[k], semb)

    def wait(k):
        pltpu.make_async_copy(a_hbm.at[sbuf.at[0]], bufas[k], sema).wait()
        pltpu.make_async_copy(b_hbm.at[dbuf.at[0]], bufbs[k], semb).wait()

    issue(0, 0)
    eidx = jnp.arange(16, dtype=jnp.int32)

    def chunk2(j, k, bufa, bufb):
        wait(k)
        issue(jnp.minimum(j + 1, NCHUNK - 1), 1 - k)

        def row(r, c2):
            for c in range(F // 16):
                sl = pl.ds(c * 16, 16)
                bufa[r, sl] = bufa[r, sl] + bufb[r, sl]
            return c2
        lax.fori_loop(0, CHUNK, row, 0)

        def grp(g, c2):
            si = sbuf[j, pl.ds(g * 16, 16)] * 4
            di = dbuf[j, pl.ds(g * 16, 16)] * 4
            ei = (j % DXB) * CHUNK + g * 16 + eidx
            ds0 = plsc.load_gather(xtab, [di]) - plsc.load_gather(xtab, [si])
            ds1 = (plsc.load_gather(xtab, [di + 1])
                   - plsc.load_gather(xtab, [si + 1]))
            ds2c = (plsc.load_gather(xtab, [di + 2])
                    - plsc.load_gather(xtab, [si + 2]))
            plsc.store_scatter(dxbuf, [ei],
                               ds0 * ds0 + ds1 * ds1 + ds2c * ds2c)
            return c2
        lax.fori_loop(0, CHUNK // 16, grp, 0)
        pltpu.sync_copy(bufa, ef_hbm.at[pl.ds(wid * PER_TILE + j * CHUNK, CHUNK)])

        @pl.when(j % DXB == DXB - 1)
        def _():
            jb = j - (DXB - 1)
            pltpu.sync_copy(
                dxbuf,
                dx_hbm.at[pl.ds(wid * PER_TILE + jb * CHUNK, DXB * CHUNK)])

    def chunk(j, carry):
        @pl.when(j % 2 == 0)
        def _():
            chunk2(j, 0, bufa0, bufb0)

        @pl.when(j % 2 == 1)
        def _():
            chunk2(j, 1, bufa1, bufb1)
        return carry
    lax.fori_loop(0, NCHUNK, chunk, 0)
    wait(NCHUNK % 2)


def _gather(a_tab, b_tab, xpad, src3d, dst3d):
    mesh = plsc.VectorSubcoreMesh(core_axis_name="c", subcore_axis_name="s")
    fn = pl.kernel(
        _gather_body,
        out_type=[
            jax.ShapeDtypeStruct((E, F), jnp.float32),
            jax.ShapeDtypeStruct((E,), jnp.float32),
        ],
        mesh=mesh,
        scratch_types=[
            pltpu.VMEM((NCHUNK, CHUNK), jnp.int32),
            pltpu.VMEM((NCHUNK, CHUNK), jnp.int32),
            pltpu.VMEM((CHUNK, F), jnp.float32),
            pltpu.VMEM((CHUNK, F), jnp.float32),
            pltpu.VMEM((CHUNK, F), jnp.float32),
            pltpu.VMEM((CHUNK, F), jnp.float32),
            pltpu.VMEM((N * 4,), jnp.float32),
            pltpu.VMEM((DXB * CHUNK,), jnp.float32),
            pltpu.SemaphoreType.DMA,
            pltpu.SemaphoreType.DMA,
        ],
        compiler_params=pltpu.CompilerParams(needs_layout_passes=False),
    )
    return fn(a_tab, b_tab, xpad, src3d, dst3d)


def _edge_body(ef_ref, d2_ref, we1dc_ref, be1_ref, we2_ref, be2_ref,
               ww1_ref, bw1_ref, ww2c_ref, bw2_ref, m_ref, wgt_ref):
    ef = ef_ref[...]
    d_row = jnp.sqrt(d2_ref[...])
    douter = jnp.dot(we1dc_ref[...], d_row,
                     preferred_element_type=jnp.float32)
    pre1 = ef + jnp.transpose(douter) + be1_ref[...]
    h1 = _silu(pre1)
    m = _silu(jnp.dot(h1, we2_ref[...], preferred_element_type=jnp.float32)
              + be2_ref[...])
    t = _silu(jnp.dot(m, ww1_ref[...], preferred_element_type=jnp.float32)
              + bw1_ref[...])
    wgt_col = jnp.dot(t, ww2c_ref[...],
                      preferred_element_type=jnp.float32) + bw2_ref[...]
    m_ref[...] = m
    wgt_ref[...] = jnp.transpose(wgt_col)


def _edge_mlp(ef, d2row, we1d_col, be1, we2, be2, ww1, bw1, ww2_col, bw2):
    be = 1280
    return pl.pallas_call(
        _edge_body,
        grid=(E // be,),
        in_specs=[
            pl.BlockSpec((be, F), lambda i: (i, 0)),
            pl.BlockSpec((1, be), lambda i: (0, i)),
            pl.BlockSpec((F, 1), lambda i: (0, 0)),
            pl.BlockSpec((1, F), lambda i: (0, 0)),
            pl.BlockSpec((F, F), lambda i: (0, 0)),
            pl.BlockSpec((1, F), lambda i: (0, 0)),
            pl.BlockSpec((F, F), lambda i: (0, 0)),
            pl.BlockSpec((1, F), lambda i: (0, 0)),
            pl.BlockSpec((F, 1), lambda i: (0, 0)),
            pl.BlockSpec((1, 1), lambda i: (0, 0)),
        ],
        out_specs=[
            pl.BlockSpec((be, F), lambda i: (i, 0)),
            pl.BlockSpec((1, be), lambda i: (0, i)),
        ],
        out_shape=[
            jax.ShapeDtypeStruct((E, F), jnp.float32),
            jax.ShapeDtypeStruct((1, E), jnp.float32),
        ],
    )(ef, d2row, we1d_col, be1, we2, be2, ww1, bw1, ww2_col, bw2)


def _scatter_m_body(m_hbm, src_hbm, accm_hbm, sbuf, bufm0, bufm1, accm,
                    seml, sems):
    cid = lax.axis_index("c")
    sid = lax.axis_index("s")
    wid = sid * 2 + cid
    zero16 = jnp.zeros((16,), jnp.float32)
    bufs = (bufm0, bufm1)

    def zrow(r, c2):
        for c in range(F // 16):
            bufm0[r, pl.ds(c * 16, 16)] = zero16
        return c2
    lax.fori_loop(0, CHUNK, zrow, 0)
    for k in range(NROW // CHUNK):
        pltpu.sync_copy(bufm0, accm.at[pl.ds(sid * NROW + k * CHUNK, CHUNK)])
    plsc.subcore_barrier()

    pltpu.sync_copy(src_hbm.at[wid], sbuf)

    def issue(j, k):
        pltpu.async_copy(m_hbm.at[pl.ds(wid * PER_TILE + j * CHUNK, CHUNK)],
                         bufs[k], seml)

    def wait(k):
        pltpu.make_async_copy(m_hbm.at[pl.ds(0, CHUNK)], bufs[k], seml).wait()

    def wait_scat(j, k):
        pltpu.make_async_copy(bufs[k], accm.at[sbuf.at[j]], sems).wait()

    issue(0, 0)

    def chunk2(j, k, bufm):
        wait(k)

        @pl.when(j >= 1)
        def _():
            wait_scat(j - 1, 1 - k)
        issue(jnp.minimum(j + 1, NCHUNK - 1), 1 - k)
        pltpu.async_copy(bufm, accm.at[sbuf.at[j]], sems, add=True)

    def chunk(j, carry):
        @pl.when(j % 2 == 0)
        def _():
            chunk2(j, 0, bufm0)

        @pl.when(j % 2 == 1)
        def _():
            chunk2(j, 1, bufm1)
        return carry
    lax.fori_loop(0, NCHUNK, chunk, 0)
    wait(NCHUNK % 2)
    wait_scat(NCHUNK - 1, (NCHUNK - 1) % 2)
    plsc.subcore_barrier()

    r0 = sid * NROW
    pltpu.sync_copy(accm.at[pl.ds(r0, NROW)], accm_hbm.at[cid].at[pl.ds(r0, NROW)])


def _scatter_m(m_rows, src3d):
    mesh = plsc.VectorSubcoreMesh(core_axis_name="c", subcore_axis_name="s")
    fn = pl.kernel(
        _scatter_m_body,
        out_type=jax.ShapeDtypeStruct((2, NPAD, F), jnp.float32),
        mesh=mesh,
        scratch_types=[
            pltpu.VMEM((NCHUNK, CHUNK), jnp.int32),
            pltpu.VMEM((CHUNK, F), jnp.float32),
            pltpu.VMEM((CHUNK, F), jnp.float32),
            pltpu.VMEM_SHARED((NPAD, F), jnp.float32),
            pltpu.SemaphoreType.DMA,
            pltpu.SemaphoreType.DMA,
        ],
        compiler_params=pltpu.CompilerParams(needs_layout_passes=False),
    )
    return fn(m_rows, src3d)


NXB = 5
NBATCH = NCHUNK // NXB


def _scatter_x_body(wgt_hbm, src_hbm, dst_hbm, xpad_hbm, accx_hbm,
                    sbuf, dbuf, xtab, bufw0, bufw1, accx, seml):
    wid = lax.axis_index("s") * 2 + lax.axis_index("c")
    zero16 = jnp.zeros((16,), jnp.float32)
    one16 = jnp.ones((16,), jnp.float32)
    bufs = (bufw0, bufw1)

    def zx(i, c2):
        accx[pl.ds(i * 16, 16)] = zero16
        return c2
    lax.fori_loop(0, NPAD * 4 // 16, zx, 0)

    pltpu.sync_copy(src_hbm.at[wid], sbuf)
    pltpu.sync_copy(dst_hbm.at[wid], dbuf)
    pltpu.sync_copy(xpad_hbm, xtab)
    eidx = jnp.arange(16, dtype=jnp.int32)

    def issue(b, k):
        base = wid * PER_TILE + b * NXB * CHUNK
        pltpu.async_copy(wgt_hbm.at[pl.ds(base, NXB * CHUNK)], bufs[k], seml)

    def wait(k):
        pltpu.make_async_copy(wgt_hbm.at[pl.ds(0, NXB * CHUNK)],
                              bufs[k], seml).wait()

    issue(0, 0)

    def batch2(b, k, bufw):
        wait(k)
        issue(jnp.minimum(b + 1, NBATCH - 1), 1 - k)

        def grp(g, c2):
            j = b * NXB + g // (CHUNK // 16)
            gg = g % (CHUNK // 16)
            ni = sbuf[j, pl.ds(gg * 16, 16)]
            si = ni * 4
            di = dbuf[j, pl.ds(gg * 16, 16)] * 4
            wv = bufw[pl.ds(g * 16, 16)]
            for c in range(3):
                xs = plsc.load_gather(xtab, [si + c])
                xd = plsc.load_gather(xtab, [di + c])
                plsc.addupdate_scatter(accx, [ni + c * NPAD], (xd - xs) * wv)
            plsc.addupdate_scatter(accx, [ni + 3 * NPAD], one16)
            return c2
        lax.fori_loop(0, NXB * CHUNK // 16, grp, 0)

    def batch(b, carry):
        @pl.when(b % 2 == 0)
        def _():
            batch2(b, 0, bufw0)

        @pl.when(b % 2 == 1)
        def _():
            batch2(b, 1, bufw1)
        return carry
    lax.fori_loop(0, NBATCH, batch, 0)
    wait(NBATCH % 2)
    pltpu.sync_copy(accx, accx_hbm.at[wid])


def _scatter_x(wgt_flat, src3d, dst3d, xpad):
    mesh = plsc.VectorSubcoreMesh(core_axis_name="c", subcore_axis_name="s")
    fn = pl.kernel(
        _scatter_x_body,
        out_type=jax.ShapeDtypeStruct((NW, NPAD * 4), jnp.float32),
        mesh=mesh,
        scratch_types=[
            pltpu.VMEM((NCHUNK, CHUNK), jnp.int32),
            pltpu.VMEM((NCHUNK, CHUNK), jnp.int32),
            pltpu.VMEM((N * 4,), jnp.float32),
            pltpu.VMEM((NXB * CHUNK,), jnp.float32),
            pltpu.VMEM((NXB * CHUNK,), jnp.float32),
            pltpu.VMEM((NPAD * 4,), jnp.float32),
            pltpu.SemaphoreType.DMA,
        ],
        compiler_params=pltpu.CompilerParams(needs_layout_passes=False),
    )
    return fn(wgt_flat, src3d, dst3d, xpad)


def _node_body(accm_ref, accx_ref, x_ref, h_ref, hn_ref, wn1b_ref, bn1_ref,
               wn2_ref, bn2_ref, xp_ref, hp_ref):
    am = accm_ref[...]
    m_sum = am[0] + am[1]
    ax = jnp.sum(accx_ref[...], axis=0)
    x_sum = jnp.transpose(ax[:3])
    cnt = jnp.transpose(ax[3:4])
    m_i = m_sum / jnp.maximum(cnt, 1.0)
    pre = (hn_ref[...]
           + jnp.dot(m_i, wn1b_ref[...], preferred_element_type=jnp.float32)
           + bn1_ref[...])
    upd = jnp.dot(_silu(pre), wn2_ref[...], preferred_element_type=jnp.float32)
    hp_ref[...] = h_ref[...] + upd + bn2_ref[...]
    xp_ref[...] = x_ref[...] + x_sum


def _node(accm, accx4, x, h, hn, wn1b, bn1, wn2, bn2):
    bn = 1024
    return pl.pallas_call(
        _node_body,
        grid=(NPAD // bn,),
        in_specs=[
            pl.BlockSpec((2, bn, F), lambda i: (0, i, 0)),
            pl.BlockSpec((NW, 4, bn), lambda i: (0, 0, i)),
            pl.BlockSpec((bn, 3), lambda i: (i, 0)),
            pl.BlockSpec((bn, F), lambda i: (i, 0)),
            pl.BlockSpec((bn, F), lambda i: (i, 0)),
            pl.BlockSpec((F, F), lambda i: (0, 0)),
            pl.BlockSpec((1, F), lambda i: (0, 0)),
            pl.BlockSpec((F, F), lambda i: (0, 0)),
            pl.BlockSpec((1, F), lambda i: (0, 0)),
        ],
        out_specs=[
            pl.BlockSpec((bn, 3), lambda i: (i, 0)),
            pl.BlockSpec((bn, F), lambda i: (i, 0)),
        ],
        out_shape=[
            jax.ShapeDtypeStruct((N, 3), jnp.float32),
            jax.ShapeDtypeStruct((N, F), jnp.float32),
        ],
    )(accm, accx4, x, h, hn, wn1b, bn1, wn2, bn2)


def kernel(x, h, edges, We1, be1, We2, be2, Ww1, bw1, Ww2, bw2, Wn1, bn1, Wn2, bn2):
    src3d = edges[0].astype(jnp.int32).reshape(NW, NCHUNK, CHUNK)
    dst3d = edges[1].astype(jnp.int32).reshape(NW, NCHUNK, CHUNK)
    xpad = jnp.pad(x, ((0, 0), (0, 1))).reshape(N * 4)
    we1a = We1[:F]
    we1b = We1[F:2 * F]
    we1d_col = We1[2 * F:2 * F + 1].reshape(F, 1)
    wn1a = Wn1[:F]
    wn1b = Wn1[F:]

    a_tab, b_tab, hn = _prep(h, we1a, we1b, wn1a)
    ef, d2 = _gather(a_tab, b_tab, xpad, src3d, dst3d)
    m_rows, wgt_row = _edge_mlp(ef, d2.reshape(1, E), we1d_col,
                                be1.reshape(1, F), We2,
                                be2.reshape(1, F), Ww1, bw1.reshape(1, F),
                                Ww2, bw2.reshape(1, 1))
    accm = _scatter_m(m_rows, src3d)
    accx = _scatter_x(wgt_row.reshape(E), src3d, dst3d, xpad)
    accx4 = accx.reshape(NW, 4, NPAD)
    x_prime, h_prime = _node(accm, accx4, x, h, hn, wn1b, bn1.reshape(1, F),
                             Wn2, bn2.reshape(1, F))
    return (x_prime, h_prime)


# speedup vs baseline: 1.9258x; 1.1246x over previous
import jax
import jax.numpy as jnp
from jax import lax
from jax.experimental import pallas as pl
from jax.experimental.pallas import tpu as pltpu
from jax.experimental.pallas import tpu_sc as plsc

N = 10000
E = 320000
F = 128
M = 128
NW = 32
PER_TILE = E // NW
CHUNK = 80
NCHUNK = PER_TILE // CHUNK
NPAD = 10240
NROW = NPAD // 16
XW = 16


def _silu(v):
    return v / (1.0 + jnp.exp(-v))


def _prep_body(h_ref, we1a_ref, we1b_ref, wn1a_ref, a_ref, b_ref, hn_ref):
    hb = h_ref[...]
    a_ref[...] = jnp.dot(hb, we1a_ref[...], preferred_element_type=jnp.float32)
    b_ref[...] = jnp.dot(hb, we1b_ref[...], preferred_element_type=jnp.float32)
    hn_ref[...] = jnp.dot(hb, wn1a_ref[...], preferred_element_type=jnp.float32)


def _prep(h, we1a, we1b, wn1a):
    bn = 1000
    return pl.pallas_call(
        _prep_body,
        grid=(N // bn,),
        in_specs=[
            pl.BlockSpec((bn, F), lambda i: (i, 0)),
            pl.BlockSpec((F, F), lambda i: (0, 0)),
            pl.BlockSpec((F, F), lambda i: (0, 0)),
            pl.BlockSpec((F, F), lambda i: (0, 0)),
        ],
        out_specs=[
            pl.BlockSpec((bn, F), lambda i: (i, 0)),
            pl.BlockSpec((bn, F), lambda i: (i, 0)),
            pl.BlockSpec((bn, F), lambda i: (i, 0)),
        ],
        out_shape=[
            jax.ShapeDtypeStruct((N, F), jnp.float32),
            jax.ShapeDtypeStruct((N, F), jnp.float32),
            jax.ShapeDtypeStruct((N, F), jnp.float32),
        ],
    )(h, we1a, we1b, wn1a)


DXB = 5


def _gather_body(a_hbm, b_hbm, xpad_hbm, src_hbm, dst_hbm, ef_hbm, dx_hbm,
                 sbuf, dbuf, bufa0, bufa1, bufb0, bufb1, xtab, dxbuf,
                 sema, semb):
    wid = lax.axis_index("s") * 2 + lax.axis_index("c")
    pltpu.sync_copy(src_hbm.at[wid], sbuf)
    pltpu.sync_copy(dst_hbm.at[wid], dbuf)
    pltpu.sync_copy(xpad_hbm, xtab)
    bufas = (bufa0, bufa1)
    bufbs = (bufb0, bufb1)

    def issue(j, k):
        pltpu.async_copy(a_hbm.at[sbuf.at[j]], bufas[k], sema)
        pltpu.async_copy(b_hbm.at[dbuf.at[j]], bufbs[k], semb)

    def wait(k):
        pltpu.make_async_copy(a_hbm.at[sbuf.at[0]], bufas[k], sema).wait()
        pltpu.make_async_copy(b_hbm.at[dbuf.at[0]], bufbs[k], semb).wait()

    issue(0, 0)
    eidx = jnp.arange(16, dtype=jnp.int32)

    def chunk2(j, k, bufa, bufb):
        wait(k)
        issue(jnp.minimum(j + 1, NCHUNK - 1), 1 - k)

        @plsc.parallel_loop(0, CHUNK, unroll=4)
        def row(r):
            for c in range(F // 16):
                sl = pl.ds(c * 16, 16)
                bufa[r, sl] = bufa[r, sl] + bufb[r, sl]

        def grp(g, c2):
            si = sbuf[j, pl.ds(g * 16, 16)] * 4
            di = dbuf[j, pl.ds(g * 16, 16)] * 4
            ei = (j % DXB) * CHUNK + g * 16 + eidx
            ds0 = plsc.load_gather(xtab, [di]) - plsc.load_gather(xtab, [si])
            ds1 = (plsc.load_gather(xtab, [di + 1])
                   - plsc.load_gather(xtab, [si + 1]))
            ds2c = (plsc.load_gather(xtab, [di + 2])
                    - plsc.load_gather(xtab, [si + 2]))
            plsc.store_scatter(dxbuf, [ei],
                               ds0 * ds0 + ds1 * ds1 + ds2c * ds2c)
            return c2
        lax.fori_loop(0, CHUNK // 16, grp, 0)
        pltpu.sync_copy(bufa, ef_hbm.at[pl.ds(wid * PER_TILE + j * CHUNK, CHUNK)])

        @pl.when(j % DXB == DXB - 1)
        def _():
            jb = j - (DXB - 1)
            pltpu.sync_copy(
                dxbuf,
                dx_hbm.at[pl.ds(wid * PER_TILE + jb * CHUNK, DXB * CHUNK)])

    def chunk(j, carry):
        @pl.when(j % 2 == 0)
        def _():
            chunk2(j, 0, bufa0, bufb0)

        @pl.when(j % 2 == 1)
        def _():
            chunk2(j, 1, bufa1, bufb1)
        return carry
    lax.fori_loop(0, NCHUNK, chunk, 0)
    wait(NCHUNK % 2)


def _gather(a_tab, b_tab, xpad, src3d, dst3d):
    mesh = plsc.VectorSubcoreMesh(core_axis_name="c", subcore_axis_name="s")
    fn = pl.kernel(
        _gather_body,
        out_type=[
            jax.ShapeDtypeStruct((E, F), jnp.float32),
            jax.ShapeDtypeStruct((E,), jnp.float32),
        ],
        mesh=mesh,
        scratch_types=[
            pltpu.VMEM((NCHUNK, CHUNK), jnp.int32),
            pltpu.VMEM((NCHUNK, CHUNK), jnp.int32),
            pltpu.VMEM((CHUNK, F), jnp.float32),
            pltpu.VMEM((CHUNK, F), jnp.float32),
            pltpu.VMEM((CHUNK, F), jnp.float32),
            pltpu.VMEM((CHUNK, F), jnp.float32),
            pltpu.VMEM((N * 4,), jnp.float32),
            pltpu.VMEM((DXB * CHUNK,), jnp.float32),
            pltpu.SemaphoreType.DMA,
            pltpu.SemaphoreType.DMA,
        ],
        compiler_params=pltpu.CompilerParams(needs_layout_passes=False),
    )
    return fn(a_tab, b_tab, xpad, src3d, dst3d)


def _edge_body(ef_ref, d2_ref, we1dc_ref, be1_ref, we2_ref, be2_ref,
               ww1_ref, bw1_ref, ww2c_ref, bw2_ref, m_ref, wgt_ref):
    ef = ef_ref[...]
    d_row = jnp.sqrt(d2_ref[...])
    douter = jnp.dot(we1dc_ref[...], d_row,
                     preferred_element_type=jnp.float32)
    pre1 = ef + jnp.transpose(douter) + be1_ref[...]
    h1 = _silu(pre1)
    m = _silu(jnp.dot(h1, we2_ref[...], preferred_element_type=jnp.float32)
              + be2_ref[...])
    t = _silu(jnp.dot(m, ww1_ref[...], preferred_element_type=jnp.float32)
              + bw1_ref[...])
    wgt_col = jnp.dot(t, ww2c_ref[...],
                      preferred_element_type=jnp.float32) + bw2_ref[...]
    m_ref[...] = m
    wgt_ref[...] = jnp.transpose(wgt_col)


def _edge_mlp(ef, d2row, we1d_col, be1, we2, be2, ww1, bw1, ww2_col, bw2):
    be = 2560
    return pl.pallas_call(
        _edge_body,
        grid=(E // be,),
        in_specs=[
            pl.BlockSpec((be, F), lambda i: (i, 0)),
            pl.BlockSpec((1, be), lambda i: (0, i)),
            pl.BlockSpec((F, 1), lambda i: (0, 0)),
            pl.BlockSpec((1, F), lambda i: (0, 0)),
            pl.BlockSpec((F, F), lambda i: (0, 0)),
            pl.BlockSpec((1, F), lambda i: (0, 0)),
            pl.BlockSpec((F, F), lambda i: (0, 0)),
            pl.BlockSpec((1, F), lambda i: (0, 0)),
            pl.BlockSpec((F, 1), lambda i: (0, 0)),
            pl.BlockSpec((1, 1), lambda i: (0, 0)),
        ],
        out_specs=[
            pl.BlockSpec((be, F), lambda i: (i, 0)),
            pl.BlockSpec((1, be), lambda i: (0, i)),
        ],
        out_shape=[
            jax.ShapeDtypeStruct((E, F), jnp.float32),
            jax.ShapeDtypeStruct((1, E), jnp.float32),
        ],
    )(ef, d2row, we1d_col, be1, we2, be2, ww1, bw1, ww2_col, bw2)


def _scatter_m_body(m_hbm, src_hbm, accm_hbm, sbuf, bufm0, bufm1, accm,
                    seml, sems):
    cid = lax.axis_index("c")
    sid = lax.axis_index("s")
    wid = sid * 2 + cid
    zero16 = jnp.zeros((16,), jnp.float32)
    bufs = (bufm0, bufm1)

    def zrow(r, c2):
        for c in range(F // 16):
            bufm0[r, pl.ds(c * 16, 16)] = zero16
        return c2
    lax.fori_loop(0, CHUNK, zrow, 0)
    for k in range(NROW // CHUNK):
        pltpu.sync_copy(bufm0, accm.at[pl.ds(sid * NROW + k * CHUNK, CHUNK)])
    plsc.subcore_barrier()

    pltpu.sync_copy(src_hbm.at[wid], sbuf)

    def issue(j, k):
        pltpu.async_copy(m_hbm.at[pl.ds(wid * PER_TILE + j * CHUNK, CHUNK)],
                         bufs[k], seml)

    def wait(k):
        pltpu.make_async_copy(m_hbm.at[pl.ds(0, CHUNK)], bufs[k], seml).wait()

    def wait_scat(j, k):
        pltpu.make_async_copy(bufs[k], accm.at[sbuf.at[j]], sems).wait()

    issue(0, 0)

    def chunk2(j, k, bufm):
        wait(k)

        @pl.when(j >= 1)
        def _():
            wait_scat(j - 1, 1 - k)
        issue(jnp.minimum(j + 1, NCHUNK - 1), 1 - k)
        pltpu.async_copy(bufm, accm.at[sbuf.at[j]], sems, add=True)

    def chunk(j, carry):
        @pl.when(j % 2 == 0)
        def _():
            chunk2(j, 0, bufm0)

        @pl.when(j % 2 == 1)
        def _():
            chunk2(j, 1, bufm1)
        return carry
    lax.fori_loop(0, NCHUNK, chunk, 0)
    wait(NCHUNK % 2)
    wait_scat(NCHUNK - 1, (NCHUNK - 1) % 2)
    plsc.subcore_barrier()

    r0 = sid * NROW
    pltpu.sync_copy(accm.at[pl.ds(r0, NROW)], accm_hbm.at[cid].at[pl.ds(r0, NROW)])


def _scatter_m(m_rows, src3d):
    mesh = plsc.VectorSubcoreMesh(core_axis_name="c", subcore_axis_name="s")
    fn = pl.kernel(
        _scatter_m_body,
        out_type=jax.ShapeDtypeStruct((2, NPAD, F), jnp.float32),
        mesh=mesh,
        scratch_types=[
            pltpu.VMEM((NCHUNK, CHUNK), jnp.int32),
            pltpu.VMEM((CHUNK, F), jnp.float32),
            pltpu.VMEM((CHUNK, F), jnp.float32),
            pltpu.VMEM_SHARED((NPAD, F), jnp.float32),
            pltpu.SemaphoreType.DMA,
            pltpu.SemaphoreType.DMA,
        ],
        compiler_params=pltpu.CompilerParams(needs_layout_passes=False),
    )
    return fn(m_rows, src3d)


NXB = 5
NBATCH = NCHUNK // NXB


def _scatter_x_body(wgt_hbm, src_hbm, dst_hbm, xpad_hbm, accx_hbm,
                    sbuf, dbuf, xtab, bufw0, bufw1, accx, seml):
    wid = lax.axis_index("s") * 2 + lax.axis_index("c")
    zero16 = jnp.zeros((16,), jnp.float32)
    one16 = jnp.ones((16,), jnp.float32)
    bufs = (bufw0, bufw1)

    def zx(i, c2):
        accx[pl.ds(i * 16, 16)] = zero16
        return c2
    lax.fori_loop(0, NPAD * 4 // 16, zx, 0)

    pltpu.sync_copy(src_hbm.at[wid], sbuf)
    pltpu.sync_copy(dst_hbm.at[wid], dbuf)
    pltpu.sync_copy(xpad_hbm, xtab)
    eidx = jnp.arange(16, dtype=jnp.int32)

    def issue(b, k):
        base = wid * PER_TILE + b * NXB * CHUNK
        pltpu.async_copy(wgt_hbm.at[pl.ds(base, NXB * CHUNK)], bufs[k], seml)

    def wait(k):
        pltpu.make_async_copy(wgt_hbm.at[pl.ds(0, NXB * CHUNK)],
                              bufs[k], seml).wait()

    issue(0, 0)

    def batch2(b, k, bufw):
        wait(k)
        issue(jnp.minimum(b + 1, NBATCH - 1), 1 - k)

        def grp(g, c2):
            j = b * NXB + g // (CHUNK // 16)
            gg = g % (CHUNK // 16)
            ni = sbuf[j, pl.ds(gg * 16, 16)]
            si = ni * 4
            di = dbuf[j, pl.ds(gg * 16, 16)] * 4
            wv = bufw[pl.ds(g * 16, 16)]
            for c in range(3):
                xs = plsc.load_gather(xtab, [si + c])
                xd = plsc.load_gather(xtab, [di + c])
                plsc.addupdate_scatter(accx, [ni + c * NPAD], (xd - xs) * wv)
            plsc.addupdate_scatter(accx, [ni + 3 * NPAD], one16)
            return c2
        lax.fori_loop(0, NXB * CHUNK // 16, grp, 0)

    def batch(b, carry):
        @pl.when(b % 2 == 0)
        def _():
            batch2(b, 0, bufw0)

        @pl.when(b % 2 == 1)
        def _():
            batch2(b, 1, bufw1)
        return carry
    lax.fori_loop(0, NBATCH, batch, 0)
    wait(NBATCH % 2)
    pltpu.sync_copy(accx, accx_hbm.at[wid])


def _scatter_x(wgt_flat, src3d, dst3d, xpad):
    mesh = plsc.VectorSubcoreMesh(core_axis_name="c", subcore_axis_name="s")
    fn = pl.kernel(
        _scatter_x_body,
        out_type=jax.ShapeDtypeStruct((NW, NPAD * 4), jnp.float32),
        mesh=mesh,
        scratch_types=[
            pltpu.VMEM((NCHUNK, CHUNK), jnp.int32),
            pltpu.VMEM((NCHUNK, CHUNK), jnp.int32),
            pltpu.VMEM((N * 4,), jnp.float32),
            pltpu.VMEM((NXB * CHUNK,), jnp.float32),
            pltpu.VMEM((NXB * CHUNK,), jnp.float32),
            pltpu.VMEM((NPAD * 4,), jnp.float32),
            pltpu.SemaphoreType.DMA,
        ],
        compiler_params=pltpu.CompilerParams(needs_layout_passes=False),
    )
    return fn(wgt_flat, src3d, dst3d, xpad)


def _node_body(accm_ref, accx_ref, x_ref, h_ref, hn_ref, wn1b_ref, bn1_ref,
               wn2_ref, bn2_ref, xp_ref, hp_ref):
    am = accm_ref[...]
    m_sum = am[0] + am[1]
    ax = jnp.sum(accx_ref[...], axis=0)
    x_sum = jnp.transpose(ax[:3])
    cnt = jnp.transpose(ax[3:4])
    m_i = m_sum / jnp.maximum(cnt, 1.0)
    pre = (hn_ref[...]
           + jnp.dot(m_i, wn1b_ref[...], preferred_element_type=jnp.float32)
           + bn1_ref[...])
    upd = jnp.dot(_silu(pre), wn2_ref[...], preferred_element_type=jnp.float32)
    hp_ref[...] = h_ref[...] + upd + bn2_ref[...]
    xp_ref[...] = x_ref[...] + x_sum


def _node(accm, accx4, x, h, hn, wn1b, bn1, wn2, bn2):
    bn = 1024
    return pl.pallas_call(
        _node_body,
        grid=(NPAD // bn,),
        in_specs=[
            pl.BlockSpec((2, bn, F), lambda i: (0, i, 0)),
            pl.BlockSpec((NW, 4, bn), lambda i: (0, 0, i)),
            pl.BlockSpec((bn, 3), lambda i: (i, 0)),
            pl.BlockSpec((bn, F), lambda i: (i, 0)),
            pl.BlockSpec((bn, F), lambda i: (i, 0)),
            pl.BlockSpec((F, F), lambda i: (0, 0)),
            pl.BlockSpec((1, F), lambda i: (0, 0)),
            pl.BlockSpec((F, F), lambda i: (0, 0)),
            pl.BlockSpec((1, F), lambda i: (0, 0)),
        ],
        out_specs=[
            pl.BlockSpec((bn, 3), lambda i: (i, 0)),
            pl.BlockSpec((bn, F), lambda i: (i, 0)),
        ],
        out_shape=[
            jax.ShapeDtypeStruct((N, 3), jnp.float32),
            jax.ShapeDtypeStruct((N, F), jnp.float32),
        ],
    )(accm, accx4, x, h, hn, wn1b, bn1, wn2, bn2)


def kernel(x, h, edges, We1, be1, We2, be2, Ww1, bw1, Ww2, bw2, Wn1, bn1, Wn2, bn2):
    src3d = edges[0].astype(jnp.int32).reshape(NW, NCHUNK, CHUNK)
    dst3d = edges[1].astype(jnp.int32).reshape(NW, NCHUNK, CHUNK)
    xpad = jnp.pad(x, ((0, 0), (0, 1))).reshape(N * 4)
    we1a = We1[:F]
    we1b = We1[F:2 * F]
    we1d_col = We1[2 * F:2 * F + 1].reshape(F, 1)
    wn1a = Wn1[:F]
    wn1b = Wn1[F:]

    a_tab, b_tab, hn = _prep(h, we1a, we1b, wn1a)
    ef, d2 = _gather(a_tab, b_tab, xpad, src3d, dst3d)
    m_rows, wgt_row = _edge_mlp(ef, d2.reshape(1, E), we1d_col,
                                be1.reshape(1, F), We2,
                                be2.reshape(1, F), Ww1, bw1.reshape(1, F),
                                Ww2, bw2.reshape(1, 1))
    accm = _scatter_m(m_rows, src3d)
    accx = _scatter_x(wgt_row.reshape(E), src3d, dst3d, xpad)
    accx4 = accx.reshape(NW, 4, NPAD)
    x_prime, h_prime = _node(accm, accx4, x, h, hn, wn1b, bn1.reshape(1, F),
                             Wn2, bn2.reshape(1, F))
    return (x_prime, h_prime)
